# baseline (device time: 86073 ns/iter reference)
import functools

import jax
import jax.numpy as jnp
from jax import lax
from jax.experimental import pallas as pl
from jax.experimental.pallas import tpu as pltpu

N_DEV = 4
N_Q = 2


def kernel(x, router_W, route_idx, expert_W):
    del router_W
    n_tok, d_model = x.shape
    e_local, _, d_ff = expert_W.shape
    blk = n_tok // N_DEV
    hw = d_ff // 2
    qw = hw // N_Q

    def body(x_ref, idx_ref, w_ref, out_hbm, acc, stage_cw, stage_ccw,
             out_sems,
             rs_send_cw, rs_recv_cw, rs_send_ccw, rs_recv_ccw,
             ag_send_cw, ag_recv_cw, ag_send_ccw, ag_recv_ccw):
        my_pos = lax.axis_index("i")
        left = (my_pos - 1) % N_DEV
        right = (my_pos + 1) % N_DEV

        def rows(b):
            return pl.ds((b % N_DEV) * blk, blk)

        def cols(direction, q):
            return pl.ds(direction * hw + q * qw, qw)

        in_flight = []
        out_dmas = []

        def out_write(b, direction, q):
            r, c = rows(b), cols(direction, q)
            dma = pltpu.make_async_copy(
                acc.at[r, c], out_hbm.at[r, c], out_sems.at[len(out_dmas)]
            )
            dma.start()
            out_dmas.append(dma)

        def compute_block(b):
            xb = x_ref[rows(b), :]
            routeb = idx_ref[rows(b), :]
            out = jnp.zeros((blk, d_ff), jnp.float32)
            for el in range(e_local):
                ge = my_pos * e_local + el
                mask = (routeb == ge).astype(jnp.float32)
                out = out + jnp.dot(
                    xb * mask, w_ref[el], preferred_element_type=jnp.float32
                )
            acc[rows(b), :] = out

        def rs_msg(s, q, direction):
            if direction == 0:
                src_b, tgt, stage, ssem, rsem = (
                    my_pos - s, right, stage_cw, rs_send_cw, rs_recv_cw)
            else:
                src_b, tgt, stage, ssem, rsem = (
                    my_pos + s, left, stage_ccw, rs_send_ccw, rs_recv_ccw)
            i = s * N_Q + q
            return pltpu.make_async_remote_copy(
                src_ref=acc.at[rows(src_b), cols(direction, q)],
                dst_ref=stage.at[s, q],
                send_sem=ssem.at[i],
                recv_sem=rsem.at[i],
                device_id=(tgt,),
                device_id_type=pl.DeviceIdType.MESH,
            )

        def rs_add(s, q, direction):
            if direction == 0:
                b, stage = my_pos - 1 - s, stage_cw
            else:
                b, stage = my_pos + 1 + s, stage_ccw
            r, c = rows(b), cols(direction, q)
            acc[r, c] = acc[r, c] + stage[s, q]
            return b

        def ag_msg(s, q, direction):
            if direction == 0:
                src_b, tgt, ssem, rsem = (
                    my_pos + 1 - s, right, ag_send_cw, ag_recv_cw)
            else:
                src_b, tgt, ssem, rsem = (
                    my_pos - 1 + s, left, ag_send_ccw, ag_recv_ccw)
            i = s * N_Q + q
            ref = acc.at[rows(src_b), cols(direction, q)]
            return pltpu.make_async_remote_copy(
                src_ref=ref,
                dst_ref=ref,
                send_sem=ssem.at[i],
                recv_sem=rsem.at[i],
                device_id=(tgt,),
                device_id_type=pl.DeviceIdType.MESH,
            )

        def start(msg):
            msg.start()
            in_flight.append(msg)
            return msg

        xb0 = x_ref[rows(my_pos), :]
        route0 = idx_ref[rows(my_pos), :]
        xms = []
        for el in range(e_local):
            ge = my_pos * e_local + el
            xms.append(xb0 * (route0 == ge).astype(jnp.float32))

        barrier_sem = pltpu.get_barrier_semaphore()
        for nbr in [left, right]:
            pl.semaphore_signal(
                barrier_sem, inc=1,
                device_id=(nbr,), device_id_type=pl.DeviceIdType.MESH,
            )
        pl.semaphore_wait(barrier_sem, 2)

        rs = {}
        ag = {}
        for q in range(N_Q):
            for d in range(2):
                c0 = d * hw + q * qw
                chunk = jnp.zeros((blk, qw), jnp.float32)
                for el in range(e_local):
                    chunk = chunk + jnp.dot(
                        xms[el], w_ref[el, :, c0:c0 + qw],
                        preferred_element_type=jnp.float32,
                    )
                acc[rows(my_pos), cols(d, q)] = chunk
                rs[(0, q, d)] = start(rs_msg(0, q, d))

        compute_block(my_pos - 1)
        compute_block(my_pos + 1)

        for s in range(N_DEV - 1):
            for q in range(N_Q):
                for d in range(2):
                    rs[(s, q, d)].wait_recv()
                    b = rs_add(s, q, d)
                    if s < N_DEV - 2:
                        rs[(s + 1, q, d)] = start(rs_msg(s + 1, q, d))
                    else:
                        ag[(0, q, d)] = start(ag_msg(0, q, d))
                        out_write(b, d, q)
            if s == 0:
                compute_block(my_pos + 2)

        for s in range(N_DEV - 1):
            for q in range(N_Q):
                for d in range(2):
                    ag[(s, q, d)].wait_recv()
                    if s < N_DEV - 2:
                        ag[(s + 1, q, d)] = start(ag_msg(s + 1, q, d))
                    recv_b = (my_pos - s) if d == 0 else (my_pos + s)
                    out_write(recv_b, d, q)

        for m in in_flight:
            m.wait_send()
        for dma in out_dmas:
            dma.wait()

        @functools.partial(
            pl.run_scoped, second_barrier=pltpu.SemaphoreType.REGULAR
        )
        def _(second_barrier):
            for nbr in [left, right]:
                pl.semaphore_signal(
                    second_barrier, inc=1,
                    device_id=(nbr,), device_id_type=pl.DeviceIdType.MESH,
                )
            pl.semaphore_wait(second_barrier, 2)

    n_msgs = (N_DEV - 1) * N_Q
    n_pieces = N_DEV * N_Q * 2
    return pl.pallas_call(
        body,
        out_shape=jax.ShapeDtypeStruct((n_tok, d_ff), jnp.float32),
        in_specs=[
            pl.BlockSpec(memory_space=pltpu.VMEM),
            pl.BlockSpec(memory_space=pltpu.VMEM),
            pl.BlockSpec(memory_space=pltpu.VMEM),
        ],
        out_specs=pl.BlockSpec(memory_space=pltpu.MemorySpace.HBM),
        scratch_shapes=[
            pltpu.VMEM((n_tok, d_ff), jnp.float32),
            pltpu.VMEM((N_DEV - 1, N_Q, blk, qw), jnp.float32),
            pltpu.VMEM((N_DEV - 1, N_Q, blk, qw), jnp.float32),
            pltpu.SemaphoreType.DMA((n_pieces,)),
            pltpu.SemaphoreType.DMA((n_msgs,)),
            pltpu.SemaphoreType.DMA((n_msgs,)),
            pltpu.SemaphoreType.DMA((n_msgs,)),
            pltpu.SemaphoreType.DMA((n_msgs,)),
            pltpu.SemaphoreType.DMA((n_msgs,)),
            pltpu.SemaphoreType.DMA((n_msgs,)),
            pltpu.SemaphoreType.DMA((n_msgs,)),
            pltpu.SemaphoreType.DMA((n_msgs,)),
        ],
        compiler_params=pltpu.CompilerParams(collective_id=0),
    )(x, route_idx, expert_W)


# device time: 85757 ns/iter; 1.0037x vs baseline; 1.0037x over previous
import functools

import jax
import jax.numpy as jnp
from jax import lax
from jax.experimental import pallas as pl
from jax.experimental.pallas import tpu as pltpu

N_DEV = 4
N_Q = 2


def kernel(x, router_W, route_idx, expert_W):
    del router_W
    n_tok, d_model = x.shape
    e_local, _, d_ff = expert_W.shape
    blk = n_tok // N_DEV
    hw = d_ff // 2
    qw = hw // N_Q

    def body(x_ref, idx_ref, w_ref, out_ref, stage_cw, stage_ccw,
             rs_send_cw, rs_recv_cw, rs_send_ccw, rs_recv_ccw,
             ag_send_cw, ag_recv_cw, ag_send_ccw, ag_recv_ccw):
        my_pos = lax.axis_index("i")
        left = (my_pos - 1) % N_DEV
        right = (my_pos + 1) % N_DEV

        def rows(b):
            return pl.ds((b % N_DEV) * blk, blk)

        def cols(direction, q):
            return pl.ds(direction * hw + q * qw, qw)

        in_flight = []

        def compute_block(b):
            xb = x_ref[rows(b), :]
            routeb = idx_ref[rows(b), :]
            acc = jnp.zeros((blk, d_ff), jnp.float32)
            for el in range(e_local):
                ge = my_pos * e_local + el
                mask = (routeb == ge).astype(jnp.float32)
                acc = acc + jnp.dot(
                    xb * mask, w_ref[el], preferred_element_type=jnp.float32
                )
            out_ref[rows(b), :] = acc

        def rs_msg(s, q, direction):
            if direction == 0:
                src_b, tgt, stage, ssem, rsem = (
                    my_pos - s, right, stage_cw, rs_send_cw, rs_recv_cw)
            else:
                src_b, tgt, stage, ssem, rsem = (
                    my_pos + s, left, stage_ccw, rs_send_ccw, rs_recv_ccw)
            i = s * N_Q + q
            return pltpu.make_async_remote_copy(
                src_ref=out_ref.at[rows(src_b), cols(direction, q)],
                dst_ref=stage.at[s, q],
                send_sem=ssem.at[i],
                recv_sem=rsem.at[i],
                device_id=(tgt,),
                device_id_type=pl.DeviceIdType.MESH,
            )

        def rs_add(s, q, direction):
            if direction == 0:
                b, stage = my_pos - 1 - s, stage_cw
            else:
                b, stage = my_pos + 1 + s, stage_ccw
            r, c = rows(b), cols(direction, q)
            out_ref[r, c] = out_ref[r, c] + stage[s, q]

        def ag_msg(s, q, direction):
            if direction == 0:
                src_b, tgt, ssem, rsem = (
                    my_pos + 1 - s, right, ag_send_cw, ag_recv_cw)
            else:
                src_b, tgt, ssem, rsem = (
                    my_pos - 1 + s, left, ag_send_ccw, ag_recv_ccw)
            i = s * N_Q + q
            ref = out_ref.at[rows(src_b), cols(direction, q)]
            return pltpu.make_async_remote_copy(
                src_ref=ref,
                dst_ref=ref,
                send_sem=ssem.at[i],
                recv_sem=rsem.at[i],
                device_id=(tgt,),
                device_id_type=pl.DeviceIdType.MESH,
            )

        def start(msg):
            msg.start()
            in_flight.append(msg)
            return msg

        xb0 = x_ref[rows(my_pos), :]
        route0 = idx_ref[rows(my_pos), :]
        xms = []
        for el in range(e_local):
            ge = my_pos * e_local + el
            xms.append(xb0 * (route0 == ge).astype(jnp.float32))

        barrier_sem = pltpu.get_barrier_semaphore()
        for nbr in [left, right]:
            pl.semaphore_signal(
                barrier_sem, inc=1,
                device_id=(nbr,), device_id_type=pl.DeviceIdType.MESH,
            )
        pl.semaphore_wait(barrier_sem, 2)

        rs = {}
        ag = {}
        for q in range(N_Q):
            for d in range(2):
                c0 = d * hw + q * qw
                chunk = jnp.zeros((blk, qw), jnp.float32)
                for el in range(e_local):
                    chunk = chunk + jnp.dot(
                        xms[el], w_ref[el, :, c0:c0 + qw],
                        preferred_element_type=jnp.float32,
                    )
                out_ref[rows(my_pos), cols(d, q)] = chunk
                rs[(0, q, d)] = start(rs_msg(0, q, d))

        compute_block(my_pos - 1)
        compute_block(my_pos + 1)

        for s in range(N_DEV - 1):
            for q in range(N_Q):
                for d in range(2):
                    rs[(s, q, d)].wait_recv()
                    rs_add(s, q, d)
                    if s < N_DEV - 2:
                        rs[(s + 1, q, d)] = start(rs_msg(s + 1, q, d))
                    else:
                        ag[(0, q, d)] = start(ag_msg(0, q, d))
            if s == 0:
                compute_block(my_pos + 2)

        for s in range(N_DEV - 1):
            for q in range(N_Q):
                for d in range(2):
                    ag[(s, q, d)].wait_recv()
                    if s < N_DEV - 2:
                        ag[(s + 1, q, d)] = start(ag_msg(s + 1, q, d))

        for m in in_flight:
            m.wait_send()

        @functools.partial(
            pl.run_scoped, second_barrier=pltpu.SemaphoreType.REGULAR
        )
        def _(second_barrier):
            for nbr in [left, right]:
                pl.semaphore_signal(
                    second_barrier, inc=1,
                    device_id=(nbr,), device_id_type=pl.DeviceIdType.MESH,
                )
            pl.semaphore_wait(second_barrier, 2)

    n_msgs = (N_DEV - 1) * N_Q
    return pl.pallas_call(
        body,
        out_shape=jax.ShapeDtypeStruct((n_tok, d_ff), jnp.float32),
        in_specs=[
            pl.BlockSpec(memory_space=pltpu.VMEM),
            pl.BlockSpec(memory_space=pltpu.VMEM),
            pl.BlockSpec(memory_space=pltpu.VMEM),
        ],
        out_specs=pl.BlockSpec(memory_space=pltpu.VMEM),
        scratch_shapes=[
            pltpu.VMEM((N_DEV - 1, N_Q, blk, qw), jnp.float32),
            pltpu.VMEM((N_DEV - 1, N_Q, blk, qw), jnp.float32),
            pltpu.SemaphoreType.DMA((n_msgs,)),
            pltpu.SemaphoreType.DMA((n_msgs,)),
            pltpu.SemaphoreType.DMA((n_msgs,)),
            pltpu.SemaphoreType.DMA((n_msgs,)),
            pltpu.SemaphoreType.DMA((n_msgs,)),
            pltpu.SemaphoreType.DMA((n_msgs,)),
            pltpu.SemaphoreType.DMA((n_msgs,)),
            pltpu.SemaphoreType.DMA((n_msgs,)),
        ],
        compiler_params=pltpu.CompilerParams(collective_id=0),
    )(x, route_idx, expert_W)
